# Initial kernel scaffold; baseline (speedup 1.0000x reference)
import functools

import jax
import jax.numpy as jnp
from jax import lax
from jax.experimental import pallas as pl
from jax.experimental.pallas import tpu as pltpu
from jax.experimental.pallas import tpu_sc as plsc

DIM = 64
B = 16384
NEG = 20
NC, NS = 2, 16
NW = NC * NS
NB = B // NW
NNEG = NB * NEG
CHUNK_B = 16
CHUNK = CHUNK_B * NEG
NCHUNK = NNEG // CHUNK
NPAIR = NCHUNK // 2
GR = 128


def _row_dot(a_ref, a_row, b_ref, b_row):
    acc = a_ref[a_row, pl.ds(0, 16)] * b_ref[b_row, pl.ds(0, 16)]
    for t in range(1, DIM // 16):
        acc = acc + a_ref[a_row, pl.ds(16 * t, 16)] * b_ref[b_row, pl.ds(16 * t, 16)]
    return jnp.sum(acc)


def _sc_body(center_hbm, context_hbm, negflat_hbm, embc_hbm, embx_hbm,
             pos_out, neg_out,
             idx_c, idx_o, idx_n, vc, uo, ns0, ns1, pos_s, negs0, negs1,
             semP, semA, semB):
    wid = lax.axis_index("s") * NC + lax.axis_index("c")
    base = wid * NB
    nbase = wid * NNEG

    pltpu.sync_copy(center_hbm.at[pl.ds(base, NB)], idx_c)
    pltpu.sync_copy(context_hbm.at[pl.ds(base, NB)], idx_o)
    pltpu.sync_copy(negflat_hbm.at[pl.ds(nbase, NNEG)], idx_n)

    copies = []
    for t in range(NB // GR):
        sl = pl.ds(t * GR, GR)
        copies.append(pltpu.async_copy(embc_hbm.at[idx_c.at[sl]], vc.at[sl], semP))
        copies.append(pltpu.async_copy(embx_hbm.at[idx_o.at[sl]], uo.at[sl], semP))

    def start_chunk_gather(ck, buf, sem):
        off = ck * CHUNK
        for o, l in ((0, GR), (GR, GR), (2 * GR, CHUNK - 2 * GR)):
            pltpu.async_copy(embx_hbm.at[idx_n.at[pl.ds(off + o, l)]],
                             buf.at[pl.ds(o, l)], sem)

    def wait_chunk(buf, sem):
        pltpu.make_async_copy(embx_hbm.at[pl.ds(0, CHUNK)], buf, sem).wait()

    start_chunk_gather(0, ns0, semA)
    for c in copies:
        c.wait()

    def pos_body(j, carry):
        pos_s[j] = _row_dot(vc, j, uo, j)
        return carry
    lax.fori_loop(0, NB, pos_body, 0)
    pltpu.sync_copy(pos_s, pos_out.at[pl.ds(base, NB)])

    def compute_chunk(ck, buf, sbuf):
        def b_body(bb, carry):
            brow = ck * CHUNK_B + bb
            for k in range(NEG):
                r = bb * NEG + k
                sbuf[r] = _row_dot(buf, r, vc, brow)
            return carry
        lax.fori_loop(0, CHUNK_B, b_body, 0)

    def pair_body(i, carry):
        ck0 = i * 2
        start_chunk_gather(ck0 + 1, ns1, semB)
        wait_chunk(ns0, semA)
        compute_chunk(ck0, ns0, negs0)
        pltpu.sync_copy(negs0, neg_out.at[pl.ds(nbase + ck0 * CHUNK, CHUNK)])

        @pl.when(i < NPAIR - 1)
        def _():
            start_chunk_gather(ck0 + 2, ns0, semA)
        wait_chunk(ns1, semB)
        compute_chunk(ck0 + 1, ns1, negs1)
        pltpu.sync_copy(negs1, neg_out.at[pl.ds(nbase + (ck0 + 1) * CHUNK, CHUNK)])
        return carry

    lax.fori_loop(0, NPAIR, pair_body, 0)


_sc_scores = pl.kernel(
    _sc_body,
    out_type=(jax.ShapeDtypeStruct((B,), jnp.float32),
              jax.ShapeDtypeStruct((B * NEG,), jnp.float32)),
    mesh=plsc.VectorSubcoreMesh(core_axis_name="c", subcore_axis_name="s"),
    scratch_types=[
        pltpu.VMEM((NB,), jnp.int32),
        pltpu.VMEM((NB,), jnp.int32),
        pltpu.VMEM((NNEG,), jnp.int32),
        pltpu.VMEM((NB, DIM), jnp.float32),
        pltpu.VMEM((NB, DIM), jnp.float32),
        pltpu.VMEM((CHUNK, DIM), jnp.float32),
        pltpu.VMEM((CHUNK, DIM), jnp.float32),
        pltpu.VMEM((NB,), jnp.float32),
        pltpu.VMEM((CHUNK,), jnp.float32),
        pltpu.VMEM((CHUNK,), jnp.float32),
        pltpu.SemaphoreType.DMA,
        pltpu.SemaphoreType.DMA,
        pltpu.SemaphoreType.DMA,
    ],
)


def _logsig(x):
    return jnp.minimum(x, 0.0) - jnp.log1p(jnp.exp(-jnp.abs(x)))


def _loss_body(pos_ref, neg_ref, out_ref):
    loss = -jnp.sum(_logsig(pos_ref[...])) - jnp.sum(_logsig(-neg_ref[...]))
    out_ref[0, 0] = loss


_tc_loss = pl.pallas_call(
    _loss_body,
    out_shape=jax.ShapeDtypeStruct((1, 1), jnp.float32),
    out_specs=pl.BlockSpec(memory_space=pltpu.SMEM),
)


def kernel(center_word, context_word, negative_samples, emb_center, emb_context):
    neg_flat = negative_samples.reshape(-1)
    pos_s, neg_s = _sc_scores(center_word, context_word, neg_flat,
                              emb_center, emb_context)
    loss = _tc_loss(pos_s.reshape(B // 128, 128), neg_s.reshape(B * NEG // 128, 128))
    return loss[0, 0]


# speedup vs baseline: 3.8205x; 3.8205x over previous
import functools

import jax
import jax.numpy as jnp
from jax import lax
from jax.experimental import pallas as pl
from jax.experimental.pallas import tpu as pltpu
from jax.experimental.pallas import tpu_sc as plsc

DIM = 64
B = 16384
NEG = 20
NC, NS = 2, 16
NW = NC * NS
NB = B // NW
NNEG = NB * NEG
CHUNK_B = 16
CHUNK = CHUNK_B * NEG
NCHUNK = NNEG // CHUNK
NPAIR = NCHUNK // 2
GR = 128


def _dot16(a_ref, a_rows, b_ref, b_rows):
    dvec = jnp.zeros((16,), jnp.int32)
    acc = jnp.zeros((16,), jnp.float32)
    for _ in range(DIM):
        acc = acc + (plsc.load_gather(a_ref, [a_rows, dvec])
                     * plsc.load_gather(b_ref, [b_rows, dvec]))
        dvec = dvec + 1
    return acc


def _sc_body(center_hbm, context_hbm, negflat_hbm, embc_hbm, embx_hbm,
             pos_out, neg_out,
             idx_c, idx_o, idx_n, vc, uo, ns0, ns1, pos_s, negs0, negs1,
             semP, semA, semB):
    wid = lax.axis_index("s") * NC + lax.axis_index("c")
    base = wid * NB
    nbase = wid * NNEG

    pltpu.sync_copy(center_hbm.at[pl.ds(base, NB)], idx_c)
    pltpu.sync_copy(context_hbm.at[pl.ds(base, NB)], idx_o)
    pltpu.sync_copy(negflat_hbm.at[pl.ds(nbase, NNEG)], idx_n)

    copies = []
    for t in range(NB // GR):
        sl = pl.ds(t * GR, GR)
        copies.append(pltpu.async_copy(embc_hbm.at[idx_c.at[sl]], vc.at[sl], semP))
        copies.append(pltpu.async_copy(embx_hbm.at[idx_o.at[sl]], uo.at[sl], semP))

    def start_chunk_gather(ck, buf, sem):
        off = ck * CHUNK
        for o, l in ((0, GR), (GR, GR), (2 * GR, CHUNK - 2 * GR)):
            pltpu.async_copy(embx_hbm.at[idx_n.at[pl.ds(off + o, l)]],
                             buf.at[pl.ds(o, l)], sem)

    def wait_chunk(buf, sem):
        pltpu.make_async_copy(embx_hbm.at[pl.ds(0, CHUNK)], buf, sem).wait()

    start_chunk_gather(0, ns0, semA)
    for c in copies:
        c.wait()

    lane = lax.iota(jnp.int32, 16)

    def pos_body(g, carry):
        rvec = g * 16 + lane
        pos_s[pl.ds(g * 16, 16)] = _dot16(vc, rvec, uo, rvec)
        return carry
    lax.fori_loop(0, NB // 16, pos_body, 0)
    pltpu.sync_copy(pos_s, pos_out.at[pl.ds(base, NB)])

    def compute_chunk(ck, buf, sbuf):
        def g_body(g, carry):
            rvec = g * 16 + lane
            brow = (ck * CHUNK + rvec) // NEG
            sbuf[pl.ds(g * 16, 16)] = _dot16(buf, rvec, vc, brow)
            return carry
        lax.fori_loop(0, CHUNK // 16, g_body, 0)

    def pair_body(i, carry):
        ck0 = i * 2
        start_chunk_gather(ck0 + 1, ns1, semB)
        wait_chunk(ns0, semA)
        compute_chunk(ck0, ns0, negs0)
        pltpu.sync_copy(negs0, neg_out.at[pl.ds(nbase + ck0 * CHUNK, CHUNK)])

        @pl.when(i < NPAIR - 1)
        def _():
            start_chunk_gather(ck0 + 2, ns0, semA)
        wait_chunk(ns1, semB)
        compute_chunk(ck0 + 1, ns1, negs1)
        pltpu.sync_copy(negs1, neg_out.at[pl.ds(nbase + (ck0 + 1) * CHUNK, CHUNK)])
        return carry

    lax.fori_loop(0, NPAIR, pair_body, 0)


_sc_scores = pl.kernel(
    _sc_body,
    out_type=(jax.ShapeDtypeStruct((B,), jnp.float32),
              jax.ShapeDtypeStruct((B * NEG,), jnp.float32)),
    mesh=plsc.VectorSubcoreMesh(core_axis_name="c", subcore_axis_name="s"),
    scratch_types=[
        pltpu.VMEM((NB,), jnp.int32),
        pltpu.VMEM((NB,), jnp.int32),
        pltpu.VMEM((NNEG,), jnp.int32),
        pltpu.VMEM((NB, DIM), jnp.float32),
        pltpu.VMEM((NB, DIM), jnp.float32),
        pltpu.VMEM((CHUNK, DIM), jnp.float32),
        pltpu.VMEM((CHUNK, DIM), jnp.float32),
        pltpu.VMEM((NB,), jnp.float32),
        pltpu.VMEM((CHUNK,), jnp.float32),
        pltpu.VMEM((CHUNK,), jnp.float32),
        pltpu.SemaphoreType.DMA,
        pltpu.SemaphoreType.DMA,
        pltpu.SemaphoreType.DMA,
    ],
    compiler_params=pltpu.CompilerParams(needs_layout_passes=False,
                                         use_tc_tiling_on_sc=False),
)


def _logsig(x):
    return jnp.minimum(x, 0.0) - jnp.log1p(jnp.exp(-jnp.abs(x)))


def _loss_body(pos_ref, neg_ref, out_ref):
    loss = -jnp.sum(_logsig(pos_ref[...])) - jnp.sum(_logsig(-neg_ref[...]))
    out_ref[0, 0] = loss


_tc_loss = pl.pallas_call(
    _loss_body,
    out_shape=jax.ShapeDtypeStruct((1, 1), jnp.float32),
    out_specs=pl.BlockSpec(memory_space=pltpu.SMEM),
)


def kernel(center_word, context_word, negative_samples, emb_center, emb_context):
    neg_flat = negative_samples.reshape(-1)
    pos_s, neg_s = _sc_scores(center_word, context_word, neg_flat,
                              emb_center, emb_context)
    loss = _tc_loss(pos_s.reshape(B // 128, 128), neg_s.reshape(B * NEG // 128, 128))
    return loss[0, 0]
